# baseline (speedup 1.0000x reference)
import jax
import jax.numpy as jnp
from jax import lax
from jax.experimental import pallas as pl
from jax.experimental.pallas import tpu as pltpu
from jax.experimental.pallas import tpu_sc as plsc

N = 10000
NP = 10240
D = 128
E = 320000
NC = 2
NS = 16
L = 16
HALF = D // NC
EPT = E // NS
K = 128
NB = 4
NCH = 160
EPTP = NCH * K
RPT = NP // NS
SUB = 64
NSUB = RPT // SUB

_f32 = jnp.float32
_i32 = jnp.int32


def _body(poi, tar_cat, src_cat, out_c,
          acc, xa,
          eb0, eb1, eb2, eb3, rb0, rb1, rb2, rb3,
          tbuf, t2,
          sl0, sl1, sl2, sl3, sg0, sg1, sg2, sg3, ss0, ss1, ss2, ss3):
    ebufs = [eb0, eb1, eb2, eb3]
    rowbs = [rb0, rb1, rb2, rb3]
    semL = [sl0, sl1, sl2, sl3]
    semG = [sg0, sg1, sg2, sg3]
    semS = [ss0, ss1, ss2, ss3]

    c = lax.axis_index("c")
    s = lax.axis_index("s")
    cN = c * NP
    row0 = s * RPT
    cbase = s * NCH

    def zero_tbuf():
        zero = jnp.zeros((L,), _f32)

        def bd(i, _):
            for k in range(HALF // L):
                tbuf[i, pl.ds(k * L, L)] = zero
            return 0

        lax.fori_loop(0, SUB, bd, 0)

    def zero_acc_stripe():
        for sub in range(NSUB):
            pltpu.sync_copy(tbuf, acc.at[pl.ds(row0 + sub * SUB, SUB)])

    def scale_chunk(b):
        eb_, rb = ebufs[b], rowbs[b]

        def gb(g, _):
            valvec = lax.bitcast_convert_type(eb_[2, pl.ds(g * L, L)], _f32)
            for j in range(L):
                e = g * L + j
                v = valvec[j]
                loads = [rb[e, pl.ds(k * L, L)] for k in range(HALF // L)]
                prods = [x * v for x in loads]
                for k in range(HALF // L):
                    rb[e, pl.ds(k * L, L)] = prods[k]
            return 0

        lax.fori_loop(0, K // L, gb, 0)

    def spmm(cat_ref, x_sp, dst):
        def issue_loads(ci, b):
            base = (cbase + ci) * 3
            pltpu.async_copy(cat_ref.at[pl.ds(base, 3)], ebufs[b], semL[b])

        def wait_loads(b):
            pltpu.make_async_copy(cat_ref.at[pl.ds(0, 3)], ebufs[b], semL[b]).wait()

        def issue_gather(b):
            pltpu.async_copy(x_sp.at[ebufs[b].at[0]], rowbs[b], semG[b])

        def wait_gather(b):
            pltpu.make_async_copy(x_sp.at[ebufs[b].at[0]], rowbs[b], semG[b]).wait()

        def issue_scatter(b):
            pltpu.async_copy(rowbs[b], dst.at[ebufs[b].at[1]], semS[b], add=True)

        def wait_scatter(b):
            pltpu.make_async_copy(rowbs[b], dst.at[ebufs[b].at[1]], semS[b]).wait()

        issue_loads(0, 0)
        issue_loads(1, 1)
        wait_loads(0)
        issue_gather(0)

        def gb(gi, _):
            for b in range(NB):
                ci = gi * NB + b
                b1 = (b + 1) % NB
                b2 = (b + 2) % NB

                @pl.when(ci + 1 < NCH)
                def _():
                    wait_loads(b1)
                    issue_gather(b1)

                @pl.when(ci >= 2)
                def _():
                    wait_scatter(b2)

                @pl.when(ci + 2 < NCH)
                def _():
                    issue_loads(ci + 2, b2)

                wait_gather(b)
                scale_chunk(b)
                issue_scatter(b)
            return 0

        lax.fori_loop(0, NCH // NB, gb, 0)
        wait_scatter((NCH - 2) % NB)
        wait_scatter((NCH - 1) % NB)

    def post_layer(rezero, scale):
        for sub in range(NSUB):
            r0l = row0 + sub * SUB
            r0g = cN + r0l
            pltpu.sync_copy(out_c.at[pl.ds(r0g, SUB)], tbuf)
            pltpu.sync_copy(xa.at[pl.ds(r0l, SUB)], t2)

            def eb(i, _):
                for k in range(HALF // L):
                    sl = pl.ds(k * L, L)
                    tbuf[i, sl] = (tbuf[i, sl] + t2[i, sl]) * scale
                return 0

            lax.fori_loop(0, SUB, eb, 0)
            pltpu.sync_copy(tbuf, out_c.at[pl.ds(r0g, SUB)])
        if rezero:
            zero_tbuf()
            zero_acc_stripe()

    for sub in range(NSUB):
        r0l = row0 + sub * SUB
        pltpu.sync_copy(poi.at[pl.ds(cN + r0l, SUB)], tbuf)
        pltpu.sync_copy(tbuf, xa.at[pl.ds(r0l, SUB)])
        pltpu.sync_copy(tbuf, out_c.at[pl.ds(cN + r0l, SUB)])
    zero_tbuf()
    zero_acc_stripe()
    plsc.subcore_barrier()

    for layer in range(2):
        spmm(tar_cat, xa, acc)
        plsc.subcore_barrier()
        spmm(src_cat, acc, xa)
        plsc.subcore_barrier()
        post_layer(rezero=(layer == 0),
                   scale=jnp.float32(1.0 if layer == 0 else 1.0 / 3.0))
        plsc.subcore_barrier()


def _pack_edges(indices, values):
    cat = jnp.stack(
        [indices[1], indices[0], lax.bitcast_convert_type(values, _i32)], 0
    )
    cat = cat.reshape(3, NS, EPT).transpose(1, 0, 2)
    cat = jnp.pad(cat, ((0, 0), (0, 0), (0, EPTP - EPT)))
    cat = cat.reshape(NS, 3, NCH, K).transpose(0, 2, 1, 3)
    return cat.reshape(NS * NCH * 3, K)


@jax.jit
def kernel(poi_embs, src_indices, src_values, tar_indices, tar_values):
    poi_cat = poi_embs.reshape(N, NC, HALF).transpose(1, 0, 2)
    poi_cat = jnp.pad(poi_cat, ((0, 0), (0, NP - N), (0, 0))).reshape(NC * NP, HALF)
    tar_cat = _pack_edges(tar_indices, tar_values)
    src_cat = _pack_edges(src_indices, src_values)

    mesh = plsc.VectorSubcoreMesh(
        core_axis_name="c", subcore_axis_name="s", num_cores=NC, num_subcores=NS
    )
    run = pl.kernel(
        _body,
        out_type=[
            jax.ShapeDtypeStruct((NC * NP, HALF), _f32),
        ],
        mesh=mesh,
        compiler_params=pltpu.CompilerParams(use_tc_tiling_on_sc=False),
        scratch_types=[
            pltpu.VMEM_SHARED((NP, HALF), _f32),
            pltpu.VMEM_SHARED((NP, HALF), _f32),
        ]
        + [pltpu.VMEM((3, K), _i32) for _ in range(NB)]
        + [pltpu.VMEM((K, HALF), _f32) for _ in range(NB)]
        + [
            pltpu.VMEM((SUB, HALF), _f32),
            pltpu.VMEM((SUB, HALF), _f32),
        ]
        + [pltpu.SemaphoreType.DMA for _ in range(3 * NB)],
    )
    (out_c,) = run(poi_cat, tar_cat, src_cat)
    return out_c.reshape(NC, NP, HALF)[:, :N].transpose(1, 0, 2).reshape(N, D)


# speedup vs baseline: 1.0434x; 1.0434x over previous
import jax
import jax.numpy as jnp
from jax import lax
from jax.experimental import pallas as pl
from jax.experimental.pallas import tpu as pltpu
from jax.experimental.pallas import tpu_sc as plsc

N = 10000
NP = 10240
D = 128
E = 320000
NC = 2
NS = 16
L = 16
HALF = D // NC
EPT = E // NS
K = 128
NB = 4
NFULL = EPT // K
TAIL = EPT - NFULL * K
RPT = NP // NS
SUB = 64
NSUB = RPT // SUB

_f32 = jnp.float32
_i32 = jnp.int32


def _body(poi, tar_r, tar_c, tarv, src_r, src_c, srcv, out_c,
          acc, xa,
          cv0, cv1, cv2, cv3, rv0, rv1, rv2, rv3,
          vv0, vv1, vv2, vv3, rb0, rb1, rb2, rb3,
          tbuf, t2,
          sl0, sl1, sl2, sl3, sg0, sg1, sg2, sg3, ss0, ss1, ss2, ss3):
    colvs = [cv0, cv1, cv2, cv3]
    rowvs = [rv0, rv1, rv2, rv3]
    valvs = [vv0, vv1, vv2, vv3]
    rowbs = [rb0, rb1, rb2, rb3]
    semL = [sl0, sl1, sl2, sl3]
    semG = [sg0, sg1, sg2, sg3]
    semS = [ss0, ss1, ss2, ss3]

    c = lax.axis_index("c")
    s = lax.axis_index("s")
    cN = c * NP
    row0 = s * RPT
    ebase = s * EPT

    def zero_tbuf():
        zero = jnp.zeros((L,), _f32)

        def bd(i, _):
            for k in range(HALF // L):
                tbuf[i, pl.ds(k * L, L)] = zero
            return 0

        lax.fori_loop(0, SUB, bd, 0)

    def zero_acc_stripe():
        for sub in range(NSUB):
            pltpu.sync_copy(tbuf, acc.at[pl.ds(row0 + sub * SUB, SUB)])

    def scale_chunk(b):
        vb, rb = valvs[b], rowbs[b]

        def gb(g, _):
            valvec = vb[pl.ds(g * L, L)]
            for j in range(L):
                e = g * L + j
                v = valvec[j]
                loads = [rb[e, pl.ds(k * L, L)] for k in range(HALF // L)]
                prods = [x * v for x in loads]
                for k in range(HALF // L):
                    rb[e, pl.ds(k * L, L)] = prods[k]
            return 0

        lax.fori_loop(0, K // L, gb, 0)

    def spmm(row_ref, col_ref, val_ref, x_sp, dst):
        def issue_loads(ci, b):
            eb = ebase + ci * K
            pltpu.async_copy(col_ref.at[pl.ds(eb, K)], colvs[b], semL[b])
            pltpu.async_copy(row_ref.at[pl.ds(eb, K)], rowvs[b], semL[b])
            pltpu.async_copy(val_ref.at[pl.ds(eb, K)], valvs[b], semL[b])

        def wait_loads(b):
            pltpu.make_async_copy(col_ref.at[pl.ds(0, K)], colvs[b], semL[b]).wait()
            pltpu.make_async_copy(row_ref.at[pl.ds(0, K)], rowvs[b], semL[b]).wait()
            pltpu.make_async_copy(val_ref.at[pl.ds(0, K)], valvs[b], semL[b]).wait()

        def issue_gather(b):
            pltpu.async_copy(x_sp.at[colvs[b]], rowbs[b], semG[b])

        def wait_gather(b):
            pltpu.make_async_copy(x_sp.at[colvs[b]], rowbs[b], semG[b]).wait()

        def issue_scatter(b):
            pltpu.async_copy(rowbs[b], dst.at[rowvs[b]], semS[b], add=True)

        def wait_scatter(b):
            pltpu.make_async_copy(rowbs[b], dst.at[rowvs[b]], semS[b]).wait()

        issue_loads(0, 0)
        issue_loads(1, 1)
        wait_loads(0)
        issue_gather(0)

        def gb(gi, _):
            for b in range(NB):
                ci = gi * NB + b
                b1 = (b + 1) % NB
                b2 = (b + 2) % NB

                @pl.when(ci + 1 < NFULL)
                def _():
                    wait_loads(b1)
                    issue_gather(b1)

                @pl.when(ci >= 2)
                def _():
                    wait_scatter(b2)

                @pl.when(ci + 2 < NFULL)
                def _():
                    issue_loads(ci + 2, b2)

                wait_gather(b)
                scale_chunk(b)
                issue_scatter(b)
            return 0

        lax.fori_loop(0, NFULL // NB, gb, 0)
        wait_scatter((NFULL - 2) % NB)
        wait_scatter((NFULL - 1) % NB)

        zi = jnp.zeros((L,), _i32)
        zf = jnp.zeros((L,), _f32)
        for g in range(K // L):
            cv0[pl.ds(g * L, L)] = zi
            rv0[pl.ds(g * L, L)] = zi
            vv0[pl.ds(g * L, L)] = zf
        eb = ebase + NFULL * K
        pltpu.sync_copy(col_ref.at[pl.ds(eb, TAIL)], cv0.at[pl.ds(0, TAIL)])
        pltpu.sync_copy(row_ref.at[pl.ds(eb, TAIL)], rv0.at[pl.ds(0, TAIL)])
        pltpu.sync_copy(val_ref.at[pl.ds(eb, TAIL)], vv0.at[pl.ds(0, TAIL)])
        pltpu.async_copy(x_sp.at[cv0], rb0, semG[0]).wait()
        scale_chunk(0)
        pltpu.sync_copy(rb0, dst.at[rv0], add=True)

    def post_layer(rezero, scale):
        for sub in range(NSUB):
            r0l = row0 + sub * SUB
            r0g = cN + r0l
            pltpu.sync_copy(out_c.at[pl.ds(r0g, SUB)], tbuf)
            pltpu.sync_copy(xa.at[pl.ds(r0l, SUB)], t2)

            def eb(i, _):
                for k in range(HALF // L):
                    sl = pl.ds(k * L, L)
                    tbuf[i, sl] = (tbuf[i, sl] + t2[i, sl]) * scale
                return 0

            lax.fori_loop(0, SUB, eb, 0)
            pltpu.sync_copy(tbuf, out_c.at[pl.ds(r0g, SUB)])
        if rezero:
            zero_tbuf()
            zero_acc_stripe()

    for sub in range(NSUB):
        r0l = row0 + sub * SUB
        pltpu.sync_copy(poi.at[pl.ds(cN + r0l, SUB)], tbuf)
        pltpu.sync_copy(tbuf, xa.at[pl.ds(r0l, SUB)])
        pltpu.sync_copy(tbuf, out_c.at[pl.ds(cN + r0l, SUB)])
    zero_tbuf()
    zero_acc_stripe()
    plsc.subcore_barrier()

    for layer in range(2):
        spmm(tar_r, tar_c, tarv, xa, acc)
        plsc.subcore_barrier()
        spmm(src_r, src_c, srcv, acc, xa)
        plsc.subcore_barrier()
        post_layer(rezero=(layer == 0),
                   scale=jnp.float32(1.0 if layer == 0 else 1.0 / 3.0))
        plsc.subcore_barrier()


@jax.jit
def kernel(poi_embs, src_indices, src_values, tar_indices, tar_values):
    poi_cat = poi_embs.reshape(N, NC, HALF).transpose(1, 0, 2)
    poi_cat = jnp.pad(poi_cat, ((0, 0), (0, NP - N), (0, 0))).reshape(NC * NP, HALF)

    mesh = plsc.VectorSubcoreMesh(
        core_axis_name="c", subcore_axis_name="s", num_cores=NC, num_subcores=NS
    )
    run = pl.kernel(
        _body,
        out_type=[
            jax.ShapeDtypeStruct((NC * NP, HALF), _f32),
        ],
        mesh=mesh,
        compiler_params=pltpu.CompilerParams(use_tc_tiling_on_sc=False),
        scratch_types=[
            pltpu.VMEM_SHARED((NP, HALF), _f32),
            pltpu.VMEM_SHARED((NP, HALF), _f32),
        ]
        + [pltpu.VMEM((K,), _i32) for _ in range(2 * NB)]
        + [pltpu.VMEM((K,), _f32) for _ in range(NB)]
        + [pltpu.VMEM((K, HALF), _f32) for _ in range(NB)]
        + [
            pltpu.VMEM((SUB, HALF), _f32),
            pltpu.VMEM((SUB, HALF), _f32),
        ]
        + [pltpu.SemaphoreType.DMA for _ in range(3 * NB)],
    )
    (out_c,) = run(poi_cat, tar_indices[0], tar_indices[1], tar_values,
                   src_indices[0], src_indices[1], src_values)
    return out_c.reshape(NC, NP, HALF)[:, :N].transpose(1, 0, 2).reshape(N, D)
